# Initial kernel scaffold; baseline (speedup 1.0000x reference)
import jax
import jax.numpy as jnp
from jax.experimental import pallas as pl


def kernel(x, time_day, time_week):
    raise NotImplementedError("write your pallas kernel here")


# speedup vs baseline: 8.5824x; 8.5824x over previous
import jax
import jax.numpy as jnp
from jax.experimental import pallas as pl


def _body(day_ref, week_ref, td_ref, tw_ref, out_ref):
    T = td_ref.shape[0]
    N = day_ref.shape[2]
    d = day_ref[0]
    w = week_ref[0]
    d_idx = jnp.clip(d * T, 0, T - 1).astype(jnp.int32)
    w_idx = jnp.clip(w, 0, 6).astype(jnp.int32)

    iota_t = jax.lax.broadcasted_iota(jnp.int32, (T, N), 0)
    oh_d = (iota_t == d_idx).astype(jnp.float32)
    iota_w = jax.lax.broadcasted_iota(jnp.int32, (8, N), 0)
    oh_w = (iota_w == w_idx).astype(jnp.float32)

    acc = jax.lax.dot_general(td_ref[...], oh_d, (((0,), (0,)), ((), ())),
                              preferred_element_type=jnp.float32)
    acc = acc + jax.lax.dot_general(tw_ref[...], oh_w, (((0,), (0,)), ((), ())),
                                    preferred_element_type=jnp.float32)
    out_ref[0, :, :] = acc


def kernel(x, time_day, time_week):
    B, S, N, _ = x.shape
    T, F = time_day.shape
    day = x[:, -1, :, 1].reshape(B, 1, N)
    week = x[:, -1, :, 2].reshape(B, 1, N)
    tw_pad = jnp.zeros((8, F), jnp.float32).at[:7].set(time_week)

    out = pl.pallas_call(
        _body,
        grid=(B,),
        in_specs=[
            pl.BlockSpec((1, 1, N), lambda b: (b, 0, 0)),
            pl.BlockSpec((1, 1, N), lambda b: (b, 0, 0)),
            pl.BlockSpec((T, F), lambda b: (0, 0)),
            pl.BlockSpec((8, F), lambda b: (0, 0)),
        ],
        out_specs=pl.BlockSpec((1, F, N), lambda b: (b, 0, 0)),
        out_shape=jax.ShapeDtypeStruct((B, F, N), jnp.float32),
    )(day, week, time_day, tw_pad)
    return out[..., None]
